# Initial kernel scaffold; baseline (speedup 1.0000x reference)
import jax
import jax.numpy as jnp
from jax.experimental import pallas as pl


def kernel(x, params, pos_edge_index, neg_edge_index):
    raise NotImplementedError("write your pallas kernel here")


# speedup vs baseline: 1.9020x; 1.9020x over previous
import functools

import jax
import jax.numpy as jnp
from jax import lax
from jax.experimental import pallas as pl
from jax.experimental.pallas import tpu as pltpu
from jax.experimental.pallas import tpu_sc as plsc

_N = 10000
_D = 256
_H = 256
_E = 80000

_NPAD = 10240
_NSUB = 16
_NCORE = 2
_BATCH = 128
_NB = 40
_EPAD = _NSUB * _NB * _BATCH
_ROWS_PER_SUB = _NPAD // _NSUB


def _segsum_body(nc, count_chunk, x_hbm, src_hbm, dst_hbm, out_hbm,
                 src_v, dst_v, rows, fill, acc, gsem):
    cid = lax.axis_index("c")
    sid = lax.axis_index("s")

    pltpu.sync_copy(src_hbm.at[cid].at[sid], src_v)
    pltpu.sync_copy(dst_hbm.at[cid].at[sid], dst_v)

    n_out = nc + (1 if count_chunk else 0)
    base = sid * _ROWS_PER_SUB

    for c in range(n_out):
        is_cnt = count_chunk and c == nc
        fill_val = jnp.float32(0.0)

        def fill_row(i, _):
            for j in range(_BATCH // 16):
                fill[i, pl.ds(j * 16, 16)] = jnp.full((16,), fill_val)
            return _
        lax.fori_loop(0, _BATCH, fill_row, 0)
        for i in range(_ROWS_PER_SUB // _BATCH):
            pltpu.sync_copy(fill, acc.at[pl.ds(base + i * _BATCH, _BATCH)])

        if is_cnt:
            def ones_row(i, _):
                for j in range(_BATCH // 16):
                    fill[i, pl.ds(j * 16, 16)] = jnp.full((16,), 1.0)
                return _
            lax.fori_loop(0, _BATCH, ones_row, 0)

        plsc.subcore_barrier()

        def body(b, _):
            if is_cnt:
                pltpu.sync_copy(fill, acc.at[dst_v.at[b]], add=True)
            else:
                pltpu.async_copy(x_hbm.at[c].at[src_v.at[b]], rows, gsem).wait()
                pltpu.sync_copy(rows, acc.at[dst_v.at[b]], add=True)
            return _
        lax.fori_loop(0, _NB, body, 0)

        plsc.subcore_barrier()
        pltpu.sync_copy(acc.at[pl.ds(base, _ROWS_PER_SUB)],
                        out_hbm.at[cid].at[c].at[pl.ds(base, _ROWS_PER_SUB)])


@functools.lru_cache(maxsize=None)
def _make_segsum(nc, count_chunk):
    n_out = nc + (1 if count_chunk else 0)
    mesh = plsc.VectorSubcoreMesh(core_axis_name="c", subcore_axis_name="s")
    return pl.kernel(
        functools.partial(_segsum_body, nc, count_chunk),
        out_type=jax.ShapeDtypeStruct((_NCORE, n_out, _NPAD, _BATCH),
                                      jnp.float32),
        mesh=mesh,
        scratch_types=[
            pltpu.VMEM((_NB, _BATCH), jnp.int32),
            pltpu.VMEM((_NB, _BATCH), jnp.int32),
            pltpu.VMEM((_BATCH, _BATCH), jnp.float32),
            pltpu.VMEM((_BATCH, _BATCH), jnp.float32),
            pltpu.VMEM_SHARED((_NPAD, _BATCH), jnp.float32),
            pltpu.SemaphoreType.DMA,
        ],
        name=f"segsum_nc{nc}{'_cnt' if count_chunk else ''}",
    )


def _dense_first_body(agg_ref, x_ref, cnt_ref, wpl_ref, wpr_ref, wnl_ref,
                      wnr_ref, bp_ref, bn_ref, out_ref):
    aggp = jnp.concatenate([agg_ref[0, 0], agg_ref[0, 1]], axis=-1)
    aggn = jnp.concatenate([agg_ref[1, 0], agg_ref[1, 1]], axis=-1)
    x = jnp.concatenate([x_ref[0], x_ref[1]], axis=-1)
    rp = 1.0 / jnp.maximum(cnt_ref[0], 1.0)
    rn = 1.0 / jnp.maximum(cnt_ref[1], 1.0)
    zp = jnp.tanh((aggp * rp) @ wpl_ref[...] + x @ wpr_ref[...] + bp_ref[...])
    zn = jnp.tanh((aggn * rn) @ wnl_ref[...] + x @ wnr_ref[...] + bn_ref[...])
    out_ref[0] = zp[:, :_BATCH]
    out_ref[1] = zp[:, _BATCH:]
    out_ref[2] = zn[:, :_BATCH]
    out_ref[3] = zn[:, _BATCH:]


def _dense_mid_body(final, agg_ref, z_ref, cnt_ref, wpl_ref, wpr_ref, wnl_ref,
                    wnr_ref, bp_ref, bn_ref, out_ref):
    mp = jnp.concatenate([agg_ref[0, c] for c in range(4)], axis=-1)
    mn = jnp.concatenate([agg_ref[1, c] for c in range(4)], axis=-1)
    z = jnp.concatenate([z_ref[c] for c in range(4)], axis=-1)
    rp = 1.0 / jnp.maximum(cnt_ref[0], 1.0)
    rn = 1.0 / jnp.maximum(cnt_ref[1], 1.0)
    mp = mp * rp
    mn = mn * rn
    bal = jnp.concatenate([mp[:, :_H], mn[:, _H:]], axis=-1)
    unbal = jnp.concatenate([mp[:, _H:], mn[:, :_H]], axis=-1)
    zp = jnp.tanh(bal @ wpl_ref[...] + z[:, :_H] @ wpr_ref[...] + bp_ref[...])
    zn = jnp.tanh(unbal @ wnl_ref[...] + z[:, _H:] @ wnr_ref[...] + bn_ref[...])
    if final:
        out_ref[...] = jnp.concatenate([zp, zn], axis=-1)
    else:
        out_ref[0] = zp[:, :_BATCH]
        out_ref[1] = zp[:, _BATCH:]
        out_ref[2] = zn[:, :_BATCH]
        out_ref[3] = zn[:, _BATCH:]


_BLK = 1024


def _dense_first(agg, xcm, cnt, p):
    grid = (_NPAD // _BLK,)
    return pl.pallas_call(
        _dense_first_body,
        grid=grid,
        in_specs=[
            pl.BlockSpec((2, 3, _BLK, _BATCH), lambda i: (0, 0, i, 0)),
            pl.BlockSpec((2, _BLK, _BATCH), lambda i: (0, i, 0)),
            pl.BlockSpec((2, _BLK, 1), lambda i: (0, i, 0)),
            pl.BlockSpec((_D, _H), lambda i: (0, 0)),
            pl.BlockSpec((_D, _H), lambda i: (0, 0)),
            pl.BlockSpec((_D, _H), lambda i: (0, 0)),
            pl.BlockSpec((_D, _H), lambda i: (0, 0)),
            pl.BlockSpec((_H,), lambda i: (0,)),
            pl.BlockSpec((_H,), lambda i: (0,)),
        ],
        out_specs=pl.BlockSpec((4, _BLK, _BATCH), lambda i: (0, i, 0)),
        out_shape=jax.ShapeDtypeStruct((4, _NPAD, _BATCH), jnp.float32),
    )(agg, xcm, cnt, p['w_pos_l'], p['w_pos_r'], p['w_neg_l'], p['w_neg_r'],
      p['b_pos'], p['b_neg'])


def _dense_mid(agg, zcm, cnt, p, final):
    grid = (_NPAD // _BLK,)
    if final:
        out_specs = pl.BlockSpec((_BLK, 2 * _H), lambda i: (i, 0))
        out_shape = jax.ShapeDtypeStruct((_NPAD, 2 * _H), jnp.float32)
    else:
        out_specs = pl.BlockSpec((4, _BLK, _BATCH), lambda i: (0, i, 0))
        out_shape = jax.ShapeDtypeStruct((4, _NPAD, _BATCH), jnp.float32)
    return pl.pallas_call(
        functools.partial(_dense_mid_body, final),
        grid=grid,
        in_specs=[
            pl.BlockSpec((2, 4, _BLK, _BATCH), lambda i: (0, 0, i, 0)),
            pl.BlockSpec((4, _BLK, _BATCH), lambda i: (0, i, 0)),
            pl.BlockSpec((2, _BLK, 1), lambda i: (0, i, 0)),
            pl.BlockSpec((2 * _H, _H), lambda i: (0, 0)),
            pl.BlockSpec((_H, _H), lambda i: (0, 0)),
            pl.BlockSpec((2 * _H, _H), lambda i: (0, 0)),
            pl.BlockSpec((_H, _H), lambda i: (0, 0)),
            pl.BlockSpec((_H,), lambda i: (0,)),
            pl.BlockSpec((_H,), lambda i: (0,)),
        ],
        out_specs=out_specs,
        out_shape=out_shape,
    )(agg, zcm, cnt, p['w_pos_l'], p['w_pos_r'], p['w_neg_l'], p['w_neg_r'],
      p['b_pos'], p['b_neg'])


def _prep_edges(pos_edge_index, neg_edge_index):
    def one(ei):
        pad = _EPAD - _E
        src = jnp.concatenate([ei[0], jnp.zeros((pad,), jnp.int32)])
        dst = jnp.concatenate([ei[1], jnp.full((pad,), _NPAD - 1, jnp.int32)])
        return (src.reshape(_NSUB, _NB, _BATCH),
                dst.reshape(_NSUB, _NB, _BATCH))
    sp, dp = one(pos_edge_index)
    sn, dn = one(neg_edge_index)
    return jnp.stack([sp, sn]), jnp.stack([dp, dn])


def kernel(x, params, pos_edge_index, neg_edge_index):
    src, dst = _prep_edges(pos_edge_index, neg_edge_index)

    xcm = jnp.pad(x, ((0, _NPAD - _N), (0, 0)))
    xcm = xcm.reshape(_NPAD, 2, _BATCH).transpose(1, 0, 2)

    agg1 = _make_segsum(2, True)(xcm, src, dst)
    cnt = agg1[:, 2, :, :1]
    z = _dense_first(agg1, xcm, cnt, params[0])

    agg2 = _make_segsum(4, False)(z, src, dst)
    z = _dense_mid(agg2, z, cnt, params[1], final=False)

    agg3 = _make_segsum(4, False)(z, src, dst)
    out = _dense_mid(agg3, z, cnt, params[2], final=True)
    return out[:_N]


# speedup vs baseline: 2.0931x; 1.1005x over previous
import functools

import jax
import jax.numpy as jnp
from jax import lax
from jax.experimental import pallas as pl
from jax.experimental.pallas import tpu as pltpu
from jax.experimental.pallas import tpu_sc as plsc

_N = 10000
_D = 256
_H = 256
_E = 80000

_NPAD = 10240
_NSUB = 16
_NCORE = 2
_BATCH = 128
_NB = 40
_EPAD = _NSUB * _NB * _BATCH
_ROWS_PER_SUB = _NPAD // _NSUB


_NBUF = 2


def _segsum_body(nc, count_chunk, x_hbm, src_hbm, dst_hbm, out_hbm,
                 src_v, dst_v, rows, acc, gsems):
    cid = lax.axis_index("c")
    sid = lax.axis_index("s")

    pltpu.sync_copy(src_hbm.at[cid].at[sid], src_v)
    pltpu.sync_copy(dst_hbm.at[cid].at[sid], dst_v)

    n_out = nc + (1 if count_chunk else 0)
    base = sid * _ROWS_PER_SUB
    n_rounds = _NB // _NBUF

    def fill_with(val):
        def fill_row(i, carry):
            for j in range(_BATCH // 16):
                rows[1, i, pl.ds(j * 16, 16)] = jnp.full((16,), val)
            return carry
        lax.fori_loop(0, _BATCH, fill_row, 0)

    for c in range(n_out):
        is_cnt = count_chunk and c == nc

        fill_with(jnp.float32(0.0))
        for i in range(_ROWS_PER_SUB // _BATCH):
            pltpu.sync_copy(rows.at[1], acc.at[pl.ds(base + i * _BATCH, _BATCH)])

        if is_cnt:
            fill_with(jnp.float32(1.0))
        else:
            for i in range(_NBUF):
                pltpu.async_copy(x_hbm.at[c].at[src_v.at[i]],
                                 rows.at[i], gsems[i])

        plsc.subcore_barrier()

        if is_cnt:
            def cbody(b, carry):
                pltpu.sync_copy(rows.at[1], acc.at[dst_v.at[b]], add=True)
                return carry
            lax.fori_loop(0, _NB, cbody, 0)
        else:
            def body(g, carry):
                for i in range(_NBUF):
                    b = g * _NBUF + i
                    pltpu.make_async_copy(x_hbm.at[c].at[src_v.at[b]],
                                          rows.at[i], gsems[i]).wait()
                    pltpu.sync_copy(rows.at[i], acc.at[dst_v.at[b]], add=True)

                    def refill(b=b, i=i):
                        pltpu.async_copy(x_hbm.at[c].at[src_v.at[b + _NBUF]],
                                         rows.at[i], gsems[i])
                    pl.when(g < n_rounds - 1)(refill)
                return carry
            lax.fori_loop(0, n_rounds, body, 0)

        plsc.subcore_barrier()
        pltpu.sync_copy(acc.at[pl.ds(base, _ROWS_PER_SUB)],
                        out_hbm.at[cid].at[c].at[pl.ds(base, _ROWS_PER_SUB)])


@functools.lru_cache(maxsize=None)
def _make_segsum(nc, count_chunk):
    n_out = nc + (1 if count_chunk else 0)
    mesh = plsc.VectorSubcoreMesh(core_axis_name="c", subcore_axis_name="s")
    return pl.kernel(
        functools.partial(_segsum_body, nc, count_chunk),
        out_type=jax.ShapeDtypeStruct((_NCORE, n_out, _NPAD, _BATCH),
                                      jnp.float32),
        mesh=mesh,
        scratch_types=[
            pltpu.VMEM((_NB, _BATCH), jnp.int32),
            pltpu.VMEM((_NB, _BATCH), jnp.int32),
            pltpu.VMEM((_NBUF, _BATCH, _BATCH), jnp.float32),
            pltpu.VMEM_SHARED((_NPAD, _BATCH), jnp.float32),
            [pltpu.SemaphoreType.DMA] * _NBUF,
        ],
        name=f"segsum_nc{nc}{'_cnt' if count_chunk else ''}",
    )


def _dense_first_body(agg_ref, x_ref, cnt_ref, wpl_ref, wpr_ref, wnl_ref,
                      wnr_ref, bp_ref, bn_ref, out_ref):
    aggp = jnp.concatenate([agg_ref[0, 0], agg_ref[0, 1]], axis=-1)
    aggn = jnp.concatenate([agg_ref[1, 0], agg_ref[1, 1]], axis=-1)
    x = jnp.concatenate([x_ref[0], x_ref[1]], axis=-1)
    rp = 1.0 / jnp.maximum(cnt_ref[0], 1.0)
    rn = 1.0 / jnp.maximum(cnt_ref[1], 1.0)
    zp = jnp.tanh((aggp * rp) @ wpl_ref[...] + x @ wpr_ref[...] + bp_ref[...])
    zn = jnp.tanh((aggn * rn) @ wnl_ref[...] + x @ wnr_ref[...] + bn_ref[...])
    out_ref[0] = zp[:, :_BATCH]
    out_ref[1] = zp[:, _BATCH:]
    out_ref[2] = zn[:, :_BATCH]
    out_ref[3] = zn[:, _BATCH:]


def _dense_mid_body(final, agg_ref, z_ref, cnt_ref, wpl_ref, wpr_ref, wnl_ref,
                    wnr_ref, bp_ref, bn_ref, out_ref):
    mp = jnp.concatenate([agg_ref[0, c] for c in range(4)], axis=-1)
    mn = jnp.concatenate([agg_ref[1, c] for c in range(4)], axis=-1)
    z = jnp.concatenate([z_ref[c] for c in range(4)], axis=-1)
    rp = 1.0 / jnp.maximum(cnt_ref[0], 1.0)
    rn = 1.0 / jnp.maximum(cnt_ref[1], 1.0)
    mp = mp * rp
    mn = mn * rn
    bal = jnp.concatenate([mp[:, :_H], mn[:, _H:]], axis=-1)
    unbal = jnp.concatenate([mp[:, _H:], mn[:, :_H]], axis=-1)
    zp = jnp.tanh(bal @ wpl_ref[...] + z[:, :_H] @ wpr_ref[...] + bp_ref[...])
    zn = jnp.tanh(unbal @ wnl_ref[...] + z[:, _H:] @ wnr_ref[...] + bn_ref[...])
    if final:
        out_ref[...] = jnp.concatenate([zp, zn], axis=-1)
    else:
        out_ref[0] = zp[:, :_BATCH]
        out_ref[1] = zp[:, _BATCH:]
        out_ref[2] = zn[:, :_BATCH]
        out_ref[3] = zn[:, _BATCH:]


_BLK = 1024


def _dense_first(agg, xcm, cnt, p):
    grid = (_NPAD // _BLK,)
    return pl.pallas_call(
        _dense_first_body,
        grid=grid,
        in_specs=[
            pl.BlockSpec((2, 3, _BLK, _BATCH), lambda i: (0, 0, i, 0)),
            pl.BlockSpec((2, _BLK, _BATCH), lambda i: (0, i, 0)),
            pl.BlockSpec((2, _BLK, 1), lambda i: (0, i, 0)),
            pl.BlockSpec((_D, _H), lambda i: (0, 0)),
            pl.BlockSpec((_D, _H), lambda i: (0, 0)),
            pl.BlockSpec((_D, _H), lambda i: (0, 0)),
            pl.BlockSpec((_D, _H), lambda i: (0, 0)),
            pl.BlockSpec((_H,), lambda i: (0,)),
            pl.BlockSpec((_H,), lambda i: (0,)),
        ],
        out_specs=pl.BlockSpec((4, _BLK, _BATCH), lambda i: (0, i, 0)),
        out_shape=jax.ShapeDtypeStruct((4, _NPAD, _BATCH), jnp.float32),
    )(agg, xcm, cnt, p['w_pos_l'], p['w_pos_r'], p['w_neg_l'], p['w_neg_r'],
      p['b_pos'], p['b_neg'])


def _dense_mid(agg, zcm, cnt, p, final):
    grid = (_NPAD // _BLK,)
    if final:
        out_specs = pl.BlockSpec((_BLK, 2 * _H), lambda i: (i, 0))
        out_shape = jax.ShapeDtypeStruct((_NPAD, 2 * _H), jnp.float32)
    else:
        out_specs = pl.BlockSpec((4, _BLK, _BATCH), lambda i: (0, i, 0))
        out_shape = jax.ShapeDtypeStruct((4, _NPAD, _BATCH), jnp.float32)
    return pl.pallas_call(
        functools.partial(_dense_mid_body, final),
        grid=grid,
        in_specs=[
            pl.BlockSpec((2, 4, _BLK, _BATCH), lambda i: (0, 0, i, 0)),
            pl.BlockSpec((4, _BLK, _BATCH), lambda i: (0, i, 0)),
            pl.BlockSpec((2, _BLK, 1), lambda i: (0, i, 0)),
            pl.BlockSpec((2 * _H, _H), lambda i: (0, 0)),
            pl.BlockSpec((_H, _H), lambda i: (0, 0)),
            pl.BlockSpec((2 * _H, _H), lambda i: (0, 0)),
            pl.BlockSpec((_H, _H), lambda i: (0, 0)),
            pl.BlockSpec((_H,), lambda i: (0,)),
            pl.BlockSpec((_H,), lambda i: (0,)),
        ],
        out_specs=out_specs,
        out_shape=out_shape,
    )(agg, zcm, cnt, p['w_pos_l'], p['w_pos_r'], p['w_neg_l'], p['w_neg_r'],
      p['b_pos'], p['b_neg'])


def _prep_edges(pos_edge_index, neg_edge_index):
    def one(ei):
        pad = _EPAD - _E
        src = jnp.concatenate([ei[0], jnp.zeros((pad,), jnp.int32)])
        dst = jnp.concatenate([ei[1], jnp.full((pad,), _NPAD - 1, jnp.int32)])
        return (src.reshape(_NSUB, _NB, _BATCH),
                dst.reshape(_NSUB, _NB, _BATCH))
    sp, dp = one(pos_edge_index)
    sn, dn = one(neg_edge_index)
    return jnp.stack([sp, sn]), jnp.stack([dp, dn])


def kernel(x, params, pos_edge_index, neg_edge_index):
    src, dst = _prep_edges(pos_edge_index, neg_edge_index)

    xcm = jnp.pad(x, ((0, _NPAD - _N), (0, 0)))
    xcm = xcm.reshape(_NPAD, 2, _BATCH).transpose(1, 0, 2)

    agg1 = _make_segsum(2, True)(xcm, src, dst)
    cnt = agg1[:, 2, :, :1]
    z = _dense_first(agg1, xcm, cnt, params[0])

    agg2 = _make_segsum(4, False)(z, src, dst)
    z = _dense_mid(agg2, z, cnt, params[1], final=False)

    agg3 = _make_segsum(4, False)(z, src, dst)
    out = _dense_mid(agg3, z, cnt, params[2], final=True)
    return out[:_N]
